# baseline (speedup 1.0000x reference)
import jax
import jax.numpy as jnp
from jax.experimental import pallas as pl

DIM = 128
NUM_Q = 8
CB = 1024
TBLK = 1024
NH = 2
H = TBLK // NH


def _rvq_kernel(x_ref, cb_ref, cbcat_ref, cn_ref, q_ref, codes_ref, loss_ref):
    xt = x_ref[0]
    rs = [xt[:, h * H:(h + 1) * H] for h in range(NH)]
    qsums = [jnp.zeros((DIM, H), jnp.float32) for _ in range(NH)]
    loss = jnp.zeros((8, 128), dtype=jnp.float32)
    iota = jax.lax.broadcasted_iota(jnp.int32, (CB, H), 0).astype(jnp.float32)
    for q in range(NUM_Q):
        cb = cb_ref[q]
        cbcat = cbcat_ref[q]
        cnc = cn_ref[q]
        ss = [
            jax.lax.dot_general(
                cb, r, (((1,), (0,)), ((), ())),
                precision=jax.lax.Precision.DEFAULT,
                preferred_element_type=jnp.float32,
            ) for r in rs
        ]
        rns = [jnp.sum(r * r, axis=0, keepdims=True) for r in rs]
        dists = [(rn - 2.0 * s) + cnc for rn, s in zip(rns, ss)]
        ms = [jnp.min(d, axis=0, keepdims=True) for d in dists]
        maskeds = [
            jnp.where(d == m, iota, jnp.float32(CB))
            for d, m in zip(dists, ms)
        ]
        idxs = [jnp.min(mk, axis=0, keepdims=True) for mk in maskeds]
        onehots = [
            (mk == ix).astype(jnp.bfloat16) for mk, ix in zip(maskeds, idxs)
        ]
        gs = [
            jax.lax.dot_general(
                cbcat, oh, (((1,), (0,)), ((), ())),
                precision=jax.lax.Precision.DEFAULT,
                preferred_element_type=jnp.float32,
            ) for oh in onehots
        ]
        for h in range(NH):
            g = gs[h]
            quant = (g[:DIM] + g[DIM:2 * DIM]) + g[2 * DIM:]
            codes_ref[0, q:q + 1, h * H:(h + 1) * H] = idxs[h].astype(jnp.int32)
            rs[h] = rs[h] - quant
            qsums[h] = qsums[h] + quant
            rr = rs[h] * rs[h]
            part = jnp.sum(rr.reshape(8, 16, H), axis=1)
            loss = loss + jnp.sum(part.reshape(8, H // 128, 128), axis=1)

    for h in range(NH):
        sl = slice(h * H, (h + 1) * H)
        xh = xt[:, sl]
        q_ref[0, :, sl] = xh + (qsums[h] - xh)

    @pl.when(pl.program_id(0) == 0)
    def _init():
        loss_ref[...] = jnp.zeros_like(loss_ref)

    loss_ref[...] += loss


def kernel(latents, codebooks):
    B, C, T = latents.shape
    TB = T // TBLK

    def _trunc16(v):
        u = jax.lax.bitcast_convert_type(v, jnp.uint32)
        return jax.lax.bitcast_convert_type(u & jnp.uint32(0xFFFF0000),
                                            jnp.float32)

    c1 = _trunc16(codebooks)
    r1 = codebooks - c1
    c2 = _trunc16(r1)
    c3 = r1 - c2
    cbcat_t = jnp.concatenate(
        [jnp.swapaxes(c, 1, 2) for c in (c1, c2, c3)], axis=1
    ).astype(jnp.bfloat16)
    cn = jnp.sum(codebooks * codebooks, axis=-1)[..., None]
    grid = (B * TB,)
    quantized, codes, loss_sum = pl.pallas_call(
        _rvq_kernel,
        grid=grid,
        in_specs=[
            pl.BlockSpec((1, DIM, TBLK), lambda i: (i // TB, 0, i % TB)),
            pl.BlockSpec((NUM_Q, CB, DIM), lambda i: (0, 0, 0)),
            pl.BlockSpec((NUM_Q, 3 * DIM, CB), lambda i: (0, 0, 0)),
            pl.BlockSpec((NUM_Q, CB, 1), lambda i: (0, 0, 0)),
        ],
        out_specs=[
            pl.BlockSpec((1, DIM, TBLK), lambda i: (i // TB, 0, i % TB)),
            pl.BlockSpec((1, NUM_Q, TBLK), lambda i: (i // TB, 0, i % TB)),
            pl.BlockSpec((8, 128), lambda i: (0, 0)),
        ],
        out_shape=[
            jax.ShapeDtypeStruct((B, C, T), jnp.float32),
            jax.ShapeDtypeStruct((B, NUM_Q, T), jnp.int32),
            jax.ShapeDtypeStruct((8, 128), jnp.float32),
        ],
    )(latents, codebooks, cbcat_t, cn)
    denom = jnp.float32(B * T * C * NUM_Q)
    loss = jnp.sum(loss_sum) / denom
    return quantized, codes, loss, loss


# speedup vs baseline: 1.0237x; 1.0237x over previous
import jax
import jax.numpy as jnp
from jax.experimental import pallas as pl

DIM = 128
NUM_Q = 8
CB = 1024
TBLK = 1024
NH = 2
H = TBLK // NH


def _rvq_kernel(x_ref, cbb_ref, cbcat_ref, cn_ref, q_ref, codes_ref, loss_ref):
    xt = x_ref[0]
    rs = [xt[:, h * H:(h + 1) * H] for h in range(NH)]
    qsums = [jnp.zeros((DIM, H), jnp.float32) for _ in range(NH)]
    loss = jnp.zeros((8, 128), dtype=jnp.float32)
    iota = jax.lax.broadcasted_iota(jnp.int32, (CB, H), 0).astype(jnp.float32)
    for q in range(NUM_Q):
        cbcat = cbcat_ref[q]
        cnc = cn_ref[q]
        ss = [
            jax.lax.dot_general(
                cbb_ref[q], r.astype(jnp.bfloat16), (((1,), (0,)), ((), ())),
                precision=jax.lax.Precision.DEFAULT,
                preferred_element_type=jnp.float32,
            ) for r in rs
        ]
        rns = [jnp.sum(r * r, axis=0, keepdims=True) for r in rs]
        dists = [(rn - 2.0 * s) + cnc for rn, s in zip(rns, ss)]
        ms = [jnp.min(d, axis=0, keepdims=True) for d in dists]
        maskeds = [
            jnp.where(d == m, iota, jnp.float32(CB))
            for d, m in zip(dists, ms)
        ]
        idxs = [jnp.min(mk, axis=0, keepdims=True) for mk in maskeds]
        onehots = [
            (mk == ix).astype(jnp.float32) for mk, ix in zip(maskeds, idxs)
        ]
        gs = [
            jax.lax.dot_general(
                cbcat, oh, (((1,), (0,)), ((), ())),
                precision=jax.lax.Precision.DEFAULT,
                preferred_element_type=jnp.float32,
            ) for oh in onehots
        ]
        for h in range(NH):
            g = gs[h]
            quant = (g[:DIM] + g[DIM:2 * DIM]) + g[2 * DIM:]
            codes_ref[0, q:q + 1, h * H:(h + 1) * H] = idxs[h].astype(jnp.int32)
            rs[h] = rs[h] - quant
            qsums[h] = qsums[h] + quant
            rr = rs[h] * rs[h]
            part = jnp.sum(rr.reshape(8, 16, H), axis=1)
            loss = loss + jnp.sum(part.reshape(8, H // 128, 128), axis=1)

    for h in range(NH):
        sl = slice(h * H, (h + 1) * H)
        xh = xt[:, sl]
        q_ref[0, :, sl] = xh + (qsums[h] - xh)

    @pl.when(pl.program_id(0) == 0)
    def _init():
        loss_ref[...] = jnp.zeros_like(loss_ref)

    loss_ref[...] += loss


def kernel(latents, codebooks):
    B, C, T = latents.shape
    TB = T // TBLK

    def _trunc16(v):
        u = jax.lax.bitcast_convert_type(v, jnp.uint32)
        return jax.lax.bitcast_convert_type(u & jnp.uint32(0xFFFF0000),
                                            jnp.float32)

    c1 = _trunc16(codebooks)
    r1 = codebooks - c1
    c2 = _trunc16(r1)
    c3 = r1 - c2
    cbcat_t = jnp.concatenate(
        [jnp.swapaxes(c, 1, 2) for c in (c1, c2, c3)], axis=1
    )
    cn = jnp.sum(codebooks * codebooks, axis=-1)[..., None]
    grid = (B * TB,)
    quantized, codes, loss_sum = pl.pallas_call(
        _rvq_kernel,
        grid=grid,
        in_specs=[
            pl.BlockSpec((1, DIM, TBLK), lambda i: (i // TB, 0, i % TB)),
            pl.BlockSpec((NUM_Q, CB, DIM), lambda i: (0, 0, 0)),
            pl.BlockSpec((NUM_Q, 3 * DIM, CB), lambda i: (0, 0, 0)),
            pl.BlockSpec((NUM_Q, CB, 1), lambda i: (0, 0, 0)),
        ],
        out_specs=[
            pl.BlockSpec((1, DIM, TBLK), lambda i: (i // TB, 0, i % TB)),
            pl.BlockSpec((1, NUM_Q, TBLK), lambda i: (i // TB, 0, i % TB)),
            pl.BlockSpec((8, 128), lambda i: (0, 0)),
        ],
        out_shape=[
            jax.ShapeDtypeStruct((B, C, T), jnp.float32),
            jax.ShapeDtypeStruct((B, NUM_Q, T), jnp.int32),
            jax.ShapeDtypeStruct((8, 128), jnp.float32),
        ],
    )(latents, codebooks.astype(jnp.bfloat16), cbcat_t, cn)
    denom = jnp.float32(B * T * C * NUM_Q)
    loss = jnp.sum(loss_sum) / denom
    return quantized, codes, loss, loss


# speedup vs baseline: 1.0514x; 1.0271x over previous
import jax
import jax.numpy as jnp
from jax.experimental import pallas as pl

DIM = 128
NUM_Q = 8
CB = 1024
TBLK = 1024
NH = 2
H = TBLK // NH


def _rvq_kernel(x_ref, cbb_ref, cbcat_ref, cn_ref, q_ref, codes_ref, loss_ref):
    xt = x_ref[0]
    rs = [xt[:, h * H:(h + 1) * H] for h in range(NH)]
    qsums = [jnp.zeros((DIM, H), jnp.float32) for _ in range(NH)]
    loss_row = jnp.zeros((1, 128), dtype=jnp.float32)
    iota = jax.lax.broadcasted_iota(jnp.int32, (CB, H), 0).astype(jnp.float32)
    for q in range(NUM_Q):
        cbcat = cbcat_ref[q]
        cnc = cn_ref[q]
        ss = [
            jax.lax.dot_general(
                cbb_ref[q], r.astype(jnp.bfloat16), (((1,), (0,)), ((), ())),
                precision=jax.lax.Precision.DEFAULT,
                preferred_element_type=jnp.float32,
            ) for r in rs
        ]
        rns = [jnp.sum(r * r, axis=0, keepdims=True) for r in rs]
        if q > 0:
            for rn in rns:
                loss_row = loss_row + jnp.sum(
                    rn.reshape(1, H // 128, 128), axis=1)
        dists = [(rn - 2.0 * s) + cnc for rn, s in zip(rns, ss)]
        ms = [jnp.min(d, axis=0, keepdims=True) for d in dists]
        maskeds = [
            jnp.where(d == m, iota, jnp.float32(CB))
            for d, m in zip(dists, ms)
        ]
        idxs = [jnp.min(mk, axis=0, keepdims=True) for mk in maskeds]
        onehots = [
            (mk == ix).astype(jnp.float32) for mk, ix in zip(maskeds, idxs)
        ]
        gs = [
            jax.lax.dot_general(
                cbcat, oh, (((1,), (0,)), ((), ())),
                precision=jax.lax.Precision.DEFAULT,
                preferred_element_type=jnp.float32,
            ) for oh in onehots
        ]
        for h in range(NH):
            g = gs[h]
            quant = (g[:DIM] + g[DIM:2 * DIM]) + g[2 * DIM:]
            codes_ref[0, q:q + 1, h * H:(h + 1) * H] = idxs[h].astype(jnp.int32)
            rs[h] = rs[h] - quant
            qsums[h] = qsums[h] + quant
            if q == NUM_Q - 1:
                rn_last = jnp.sum(rs[h] * rs[h], axis=0, keepdims=True)
                loss_row = loss_row + jnp.sum(
                    rn_last.reshape(1, H // 128, 128), axis=1)

    for h in range(NH):
        sl = slice(h * H, (h + 1) * H)
        xh = xt[:, sl]
        q_ref[0, :, sl] = xh + (qsums[h] - xh)

    @pl.when(pl.program_id(0) == 0)
    def _init():
        loss_ref[...] = jnp.zeros_like(loss_ref)

    loss_ref[0:1, :] += loss_row


def kernel(latents, codebooks):
    B, C, T = latents.shape
    TB = T // TBLK

    def _trunc16(v):
        u = jax.lax.bitcast_convert_type(v, jnp.uint32)
        return jax.lax.bitcast_convert_type(u & jnp.uint32(0xFFFF0000),
                                            jnp.float32)

    c1 = _trunc16(codebooks)
    r1 = codebooks - c1
    c2 = _trunc16(r1)
    c3 = r1 - c2
    cbcat_t = jnp.concatenate(
        [jnp.swapaxes(c, 1, 2) for c in (c1, c2, c3)], axis=1
    )
    cn = jnp.sum(codebooks * codebooks, axis=-1)[..., None]
    grid = (B * TB,)
    quantized, codes, loss_sum = pl.pallas_call(
        _rvq_kernel,
        grid=grid,
        in_specs=[
            pl.BlockSpec((1, DIM, TBLK), lambda i: (i // TB, 0, i % TB)),
            pl.BlockSpec((NUM_Q, CB, DIM), lambda i: (0, 0, 0)),
            pl.BlockSpec((NUM_Q, 3 * DIM, CB), lambda i: (0, 0, 0)),
            pl.BlockSpec((NUM_Q, CB, 1), lambda i: (0, 0, 0)),
        ],
        out_specs=[
            pl.BlockSpec((1, DIM, TBLK), lambda i: (i // TB, 0, i % TB)),
            pl.BlockSpec((1, NUM_Q, TBLK), lambda i: (i // TB, 0, i % TB)),
            pl.BlockSpec((8, 128), lambda i: (0, 0)),
        ],
        out_shape=[
            jax.ShapeDtypeStruct((B, C, T), jnp.float32),
            jax.ShapeDtypeStruct((B, NUM_Q, T), jnp.int32),
            jax.ShapeDtypeStruct((8, 128), jnp.float32),
        ],
    )(latents, codebooks.astype(jnp.bfloat16), cbcat_t, cn)
    denom = jnp.float32(B * T * C * NUM_Q)
    loss = jnp.sum(loss_sum) / denom
    return quantized, codes, loss, loss


# speedup vs baseline: 1.0692x; 1.0170x over previous
import jax
import jax.numpy as jnp
from jax.experimental import pallas as pl

DIM = 128
NUM_Q = 8
CB = 1024
TBLK = 2048
NH = 2
H = TBLK // NH


def _rvq_kernel(x_ref, cbb_ref, cbcat_ref, cn_ref, q_ref, codes_ref, loss_ref):
    xt = x_ref[0]
    rs = [xt[:, h * H:(h + 1) * H] for h in range(NH)]
    qsums = [jnp.zeros((DIM, H), jnp.float32) for _ in range(NH)]
    loss_row = jnp.zeros((1, 128), dtype=jnp.float32)
    iota = jax.lax.broadcasted_iota(jnp.int32, (CB, H), 0).astype(jnp.float32)
    for q in range(NUM_Q):
        cbcat = cbcat_ref[q]
        cnc = cn_ref[q]
        ss = [
            jax.lax.dot_general(
                cbb_ref[q], r.astype(jnp.bfloat16), (((1,), (0,)), ((), ())),
                precision=jax.lax.Precision.DEFAULT,
                preferred_element_type=jnp.float32,
            ) for r in rs
        ]
        rns = [jnp.sum(r * r, axis=0, keepdims=True) for r in rs]
        if q > 0:
            for rn in rns:
                loss_row = loss_row + jnp.sum(
                    rn.reshape(1, H // 128, 128), axis=1)
        dists = [(rn - 2.0 * s) + cnc for rn, s in zip(rns, ss)]
        ms = [jnp.min(d, axis=0, keepdims=True) for d in dists]
        maskeds = [
            jnp.where(d == m, iota, jnp.float32(CB))
            for d, m in zip(dists, ms)
        ]
        idxs = [jnp.min(mk, axis=0, keepdims=True) for mk in maskeds]
        onehots = [
            (mk == ix).astype(jnp.float32) for mk, ix in zip(maskeds, idxs)
        ]
        gs = [
            jax.lax.dot_general(
                cbcat, oh, (((1,), (0,)), ((), ())),
                precision=jax.lax.Precision.DEFAULT,
                preferred_element_type=jnp.float32,
            ) for oh in onehots
        ]
        for h in range(NH):
            g = gs[h]
            quant = (g[:DIM] + g[DIM:2 * DIM]) + g[2 * DIM:]
            codes_ref[0, q:q + 1, h * H:(h + 1) * H] = idxs[h].astype(jnp.int32)
            rs[h] = rs[h] - quant
            qsums[h] = qsums[h] + quant
            if q == NUM_Q - 1:
                rn_last = jnp.sum(rs[h] * rs[h], axis=0, keepdims=True)
                loss_row = loss_row + jnp.sum(
                    rn_last.reshape(1, H // 128, 128), axis=1)

    for h in range(NH):
        sl = slice(h * H, (h + 1) * H)
        xh = xt[:, sl]
        q_ref[0, :, sl] = xh + (qsums[h] - xh)

    @pl.when(pl.program_id(0) == 0)
    def _init():
        loss_ref[...] = jnp.zeros_like(loss_ref)

    loss_ref[0:1, :] += loss_row


def kernel(latents, codebooks):
    B, C, T = latents.shape
    TB = T // TBLK

    def _trunc16(v):
        u = jax.lax.bitcast_convert_type(v, jnp.uint32)
        return jax.lax.bitcast_convert_type(u & jnp.uint32(0xFFFF0000),
                                            jnp.float32)

    c1 = _trunc16(codebooks)
    r1 = codebooks - c1
    c2 = _trunc16(r1)
    c3 = r1 - c2
    cbcat_t = jnp.concatenate(
        [jnp.swapaxes(c, 1, 2) for c in (c1, c2, c3)], axis=1
    )
    cn = jnp.sum(codebooks * codebooks, axis=-1)[..., None]
    grid = (B * TB,)
    quantized, codes, loss_sum = pl.pallas_call(
        _rvq_kernel,
        grid=grid,
        in_specs=[
            pl.BlockSpec((1, DIM, TBLK), lambda i: (i // TB, 0, i % TB)),
            pl.BlockSpec((NUM_Q, CB, DIM), lambda i: (0, 0, 0)),
            pl.BlockSpec((NUM_Q, 3 * DIM, CB), lambda i: (0, 0, 0)),
            pl.BlockSpec((NUM_Q, CB, 1), lambda i: (0, 0, 0)),
        ],
        out_specs=[
            pl.BlockSpec((1, DIM, TBLK), lambda i: (i // TB, 0, i % TB)),
            pl.BlockSpec((1, NUM_Q, TBLK), lambda i: (i // TB, 0, i % TB)),
            pl.BlockSpec((8, 128), lambda i: (0, 0)),
        ],
        out_shape=[
            jax.ShapeDtypeStruct((B, C, T), jnp.float32),
            jax.ShapeDtypeStruct((B, NUM_Q, T), jnp.int32),
            jax.ShapeDtypeStruct((8, 128), jnp.float32),
        ],
    )(latents, codebooks.astype(jnp.bfloat16), cbcat_t, cn)
    denom = jnp.float32(B * T * C * NUM_Q)
    loss = jnp.sum(loss_sum) / denom
    return quantized, codes, loss, loss
